# Initial kernel scaffold; baseline (speedup 1.0000x reference)
import jax
import jax.numpy as jnp
from jax.experimental import pallas as pl


def kernel(input_ids, fake_ids, item_table, pos_table, fake_table):
    raise NotImplementedError("write your pallas kernel here")


# speedup vs baseline: 1.1977x; 1.1977x over previous
import functools

import jax
import jax.numpy as jnp
from jax import lax
from jax.experimental import pallas as pl
from jax.experimental.pallas import tpu as pltpu
from jax.experimental.pallas import tpu_sc as plsc

BATCH = 4096
SEQ = 200
N = BATCH * SEQ
D_ITEM = 64
D_FAKE = 16
D_OUT = D_ITEM + D_FAKE
NUM_WORKERS = 32
PER_WORKER = N // NUM_WORKERS
CHUNK = 128
NUM_CHUNKS = PER_WORKER // CHUNK


def _sc_embed(ids, fids, item_table, pos_table, fake_table):
    mesh = plsc.VectorSubcoreMesh(core_axis_name="c", subcore_axis_name="s")

    @functools.partial(
        pl.kernel,
        mesh=mesh,
        compiler_params=pltpu.CompilerParams(use_tc_tiling_on_sc=False),
        out_type=jax.ShapeDtypeStruct((N, D_OUT), jnp.float32),
        scratch_types=[
            pltpu.VMEM((CHUNK,), jnp.int32),
            pltpu.VMEM((CHUNK,), jnp.int32),
            pltpu.VMEM((CHUNK, D_ITEM), jnp.float32),
            pltpu.VMEM((CHUNK, D_FAKE), jnp.float32),
            pltpu.VMEM((CHUNK, D_OUT), jnp.float32),
            pltpu.VMEM((SEQ, D_ITEM), jnp.float32),
            pltpu.SemaphoreType.DMA,
            pltpu.SemaphoreType.DMA,
        ],
    )
    def k(ids_hbm, fids_hbm, item_hbm, pos_hbm, fake_hbm, out_hbm,
          idx_v, fid_v, item_v, fake_v, out_v, pos_v, sem_i, sem_f):
        wid = lax.axis_index("s") * 2 + lax.axis_index("c")
        pltpu.sync_copy(pos_hbm, pos_v)

        def chunk_body(c, carry):
            base = wid * PER_WORKER + c * CHUNK
            pltpu.sync_copy(ids_hbm.at[pl.ds(base, CHUNK)], idx_v)
            pltpu.sync_copy(fids_hbm.at[pl.ds(base, CHUNK)], fid_v)
            cp_i = pltpu.async_copy(item_hbm.at[idx_v], item_v, sem_i)
            cp_f = pltpu.async_copy(fake_hbm.at[fid_v], fake_v, sem_f)
            cp_i.wait()
            cp_f.wait()

            def row_body(r, rcarry):
                p = lax.rem(base + r, SEQ)
                for j in range(D_ITEM // 16):
                    out_v[r, pl.ds(j * 16, 16)] = (
                        item_v[r, pl.ds(j * 16, 16)] + pos_v[p, pl.ds(j * 16, 16)]
                    )
                out_v[r, pl.ds(D_ITEM, 16)] = fake_v[r, :]
                return rcarry

            lax.fori_loop(0, CHUNK, row_body, 0)
            pltpu.sync_copy(out_v, out_hbm.at[pl.ds(base, CHUNK)])
            return carry

        lax.fori_loop(0, NUM_CHUNKS, chunk_body, 0)

    return k(ids, fids, item_table, pos_table, fake_table)


def kernel(input_ids, fake_ids, item_table, pos_table, fake_table):
    ids = input_ids.reshape(-1).astype(jnp.int32)
    fids = fake_ids.reshape(-1).astype(jnp.int32)
    out = _sc_embed(ids, fids, item_table, pos_table, fake_table)
    return out.reshape(BATCH, SEQ, D_OUT)
